# Initial kernel scaffold; baseline (speedup 1.0000x reference)
import jax
import jax.numpy as jnp
from jax.experimental import pallas as pl


def kernel(logits, labels):
    raise NotImplementedError("write your pallas kernel here")


# speedup vs baseline: 14.1171x; 14.1171x over previous
import functools

import jax
import jax.numpy as jnp
from jax import lax
from jax.experimental import pallas as pl
from jax.experimental.pallas import tpu as pltpu
from jax.experimental.pallas import tpu_sc as plsc

NIMG = 8
PIX = 512 * 512
NWORK = 32
PER_W = NIMG * PIX // NWORK
CHUNK = 8192
NBINS = 4096
NLANE = 16
OFF = (115 << 8) - 1
ROWS = NBINS * NLANE // 128


def _sc_hist_body(lg_hbm, lb_hbm, out_hbm, lbuf, abuf, hist):
    wid = lax.axis_index("c") * 16 + lax.axis_index("s")
    lanes = lax.iota(jnp.int32, NLANE)

    def zero(j, carry):
        hist[pl.ds(j * NLANE, NLANE)] = jnp.zeros((NLANE,), jnp.int32)
        return carry

    lax.fori_loop(0, NBINS, zero, 0)

    base = wid * PER_W
    for chunk in range(PER_W // CHUNK):
        off = base + chunk * CHUNK
        pltpu.sync_copy(lg_hbm.at[pl.ds(off, CHUNK)], lbuf)
        pltpu.sync_copy(lb_hbm.at[pl.ds(off, CHUNK)], abuf)

        def body(i, carry):
            lg = lbuf[pl.ds(i * NLANE, NLANE)]
            lb = abuf[pl.ds(i * NLANE, NLANE)]
            e = 1.0 - lg * (2.0 * lb.astype(jnp.float32) - 1.0)
            bits = lax.bitcast_convert_type(e, jnp.int32)
            bn = jnp.minimum(lax.shift_right_logical(bits, 15) - OFF,
                             NBINS - 1)
            bn = jnp.where(e < jnp.float32(2.0 ** -12),
                           jnp.zeros((NLANE,), jnp.int32), bn)
            addr = bn * NLANE + lanes
            incr = 1 + lb * 65536
            plsc.addupdate_scatter(hist, [addr], incr)
            return carry

        lax.fori_loop(0, CHUNK // NLANE, body, 0)

    pltpu.sync_copy(hist, out_hbm.at[pl.ds(base, PER_W)])


def _tc_finish_body(hist_ref, out_ref):
    x = (hist_ref[0] + hist_ref[1] + hist_ref[2] + hist_ref[3])
    cnt = jnp.bitwise_and(x, 65535).astype(jnp.float32)
    pos = lax.shift_right_logical(x, 16).astype(jnp.float32)

    k = lax.broadcasted_iota(jnp.int32, (128, 8), 0)
    b8 = lax.broadcasted_iota(jnp.int32, (128, 8), 1)
    m16 = (k // NLANE == b8).astype(jnp.float32)
    dot = functools.partial(jnp.dot, precision=jax.lax.Precision.HIGHEST,
                            preferred_element_type=jnp.float32)
    cnt_b = dot(cnt, m16)
    pos_b = dot(pos, m16)

    i8 = lax.broadcasted_iota(jnp.int32, (8, 8), 0)
    j8 = lax.broadcasted_iota(jnp.int32, (8, 8), 1)
    l8 = (i8 <= j8).astype(jnp.float32)
    rr = lax.broadcasted_iota(jnp.int32, (ROWS, ROWS), 0)
    cc = lax.broadcasted_iota(jnp.int32, (ROWS, ROWS), 1)
    estrict = (cc < rr).astype(jnp.float32)
    ones8 = jnp.ones((8, 1), jnp.float32)

    def cum(z):
        within = dot(z, l8)
        totals = dot(z, ones8)
        before = dot(estrict, totals)
        return within + before

    casc = cum(cnt_b)
    pasc = cum(pos_b)
    total = jnp.sum(cnt_b)
    gsum = jnp.sum(pos_b)

    n_b = total - casc + cnt_b
    cg_b = gsum - pasc + pos_b
    denom = jnp.maximum(gsum + n_b - cg_b, 1.0)
    jac = jnp.where(n_b > 0, 1.0 - (gsum - cg_b) / denom, 0.0)

    g = (lax.broadcasted_iota(jnp.int32, (ROWS, 8), 0) * 8
         + lax.broadcasted_iota(jnp.int32, (ROWS, 8), 1))

    def center(gg):
        u = lax.shift_left(gg + OFF, 15) | (1 << 14)
        return jnp.where(gg < 1, 0.0, lax.bitcast_convert_type(u, jnp.float32))

    da = center(g) - center(g - 1)
    s = jnp.sum(jac * da)
    out_ref[...] = jnp.full((1, 1, 128), s, jnp.float32)


def kernel(logits, labels):
    lg = logits.reshape(NWORK * PER_W).astype(jnp.float32)
    lb = labels.reshape(NWORK * PER_W)

    mesh = plsc.VectorSubcoreMesh(core_axis_name="c", subcore_axis_name="s")
    hist = pl.kernel(
        _sc_hist_body,
        mesh=mesh,
        compiler_params=pltpu.CompilerParams(needs_layout_passes=False),
        out_type=jax.ShapeDtypeStruct((NWORK * PER_W,), jnp.int32),
        scratch_types=[
            pltpu.VMEM((CHUNK,), jnp.float32),
            pltpu.VMEM((CHUNK,), jnp.int32),
            pltpu.VMEM((NBINS * NLANE,), jnp.int32),
        ],
    )(lg, lb)

    hist3 = hist.reshape(NWORK, ROWS, 128)
    per_img = pl.pallas_call(
        _tc_finish_body,
        grid=(NIMG,),
        in_specs=[pl.BlockSpec((4, ROWS, 128), lambda i: (i, 0, 0))],
        out_specs=pl.BlockSpec((1, 1, 128), lambda i: (i, 0, 0)),
        out_shape=jax.ShapeDtypeStruct((NIMG, 1, 128), jnp.float32),
    )(hist3)

    return jnp.sum(per_img[:, 0, 0]) / NIMG


# speedup vs baseline: 17.1683x; 1.2161x over previous
import functools

import jax
import jax.numpy as jnp
from jax import lax
from jax.experimental import pallas as pl
from jax.experimental.pallas import tpu as pltpu
from jax.experimental.pallas import tpu_sc as plsc

NIMG = 8
PIX = 512 * 512
NWORK = 32
PER_W = NIMG * PIX // NWORK
CHUNK = 8192
NBINS = 4096
NLANE = 16
OFF = (115 << 8) - 1
ROWS = NBINS * NLANE // 128


UNROLL = 4


def _sc_hist_body(lg_hbm, lb_hbm, out_hbm, lbuf, abuf, hist, sems):
    wid = lax.axis_index("c") * 16 + lax.axis_index("s")
    lanes = lax.iota(jnp.int32, NLANE)
    base = wid * PER_W
    nch = PER_W // CHUNK

    def issue(c, b):
        off = base + c * CHUNK
        return (
            pltpu.async_copy(lg_hbm.at[pl.ds(off, CHUNK)],
                             lbuf.at[pl.ds(b * CHUNK, CHUNK)], sems.at[b]),
            pltpu.async_copy(lb_hbm.at[pl.ds(off, CHUNK)],
                             abuf.at[pl.ds(b * CHUNK, CHUNK)], sems.at[b]),
        )

    pending = issue(0, 0)

    def zero(j, carry):
        for u in range(8):
            hist[pl.ds(j * 8 * NLANE + u * NLANE, NLANE)] = jnp.zeros(
                (NLANE,), jnp.int32)
        return carry

    lax.fori_loop(0, NBINS // 8, zero, 0)

    for c in range(nch):
        b = c & 1
        for h in pending:
            h.wait()
        if c + 1 < nch:
            pending = issue(c + 1, 1 - b)

        def body(i, carry):
            for u in range(UNROLL):
                sl = pl.ds(b * CHUNK + i * (UNROLL * NLANE) + u * NLANE,
                           NLANE)
                lg = lbuf[sl]
                lb = abuf[sl]
                e = 1.0 - lg * (2.0 * lb.astype(jnp.float32) - 1.0)
                bits = lax.bitcast_convert_type(e, jnp.int32)
                bn = jnp.minimum(lax.shift_right_logical(bits, 15) - OFF,
                                 NBINS - 1)
                bn = jnp.where(e < jnp.float32(2.0 ** -12),
                               jnp.zeros((NLANE,), jnp.int32), bn)
                addr = bn * NLANE + lanes
                incr = 1 + lb * 65536
                plsc.addupdate_scatter(hist, [addr], incr)
            return carry

        lax.fori_loop(0, CHUNK // (UNROLL * NLANE), body, 0)

    pltpu.sync_copy(hist, out_hbm.at[pl.ds(base, PER_W)])


def _tc_finish_body(hist_ref, out_ref):
    x = (hist_ref[0] + hist_ref[1] + hist_ref[2] + hist_ref[3])
    cnt = jnp.bitwise_and(x, 65535).astype(jnp.float32)
    pos = lax.shift_right_logical(x, 16).astype(jnp.float32)

    k = lax.broadcasted_iota(jnp.int32, (128, 8), 0)
    b8 = lax.broadcasted_iota(jnp.int32, (128, 8), 1)
    m16 = (k // NLANE == b8).astype(jnp.float32)
    dot = functools.partial(jnp.dot, precision=jax.lax.Precision.HIGHEST,
                            preferred_element_type=jnp.float32)
    cnt_b = dot(cnt, m16)
    pos_b = dot(pos, m16)

    i8 = lax.broadcasted_iota(jnp.int32, (8, 8), 0)
    j8 = lax.broadcasted_iota(jnp.int32, (8, 8), 1)
    l8 = (i8 <= j8).astype(jnp.float32)
    rr = lax.broadcasted_iota(jnp.int32, (ROWS, ROWS), 0)
    cc = lax.broadcasted_iota(jnp.int32, (ROWS, ROWS), 1)
    estrict = (cc < rr).astype(jnp.float32)
    ones8 = jnp.ones((8, 1), jnp.float32)

    def cum(z):
        within = dot(z, l8)
        totals = dot(z, ones8)
        before = dot(estrict, totals)
        return within + before

    casc = cum(cnt_b)
    pasc = cum(pos_b)
    total = jnp.sum(cnt_b)
    gsum = jnp.sum(pos_b)

    n_b = total - casc + cnt_b
    cg_b = gsum - pasc + pos_b
    denom = jnp.maximum(gsum + n_b - cg_b, 1.0)
    jac = jnp.where(n_b > 0, 1.0 - (gsum - cg_b) / denom, 0.0)

    g = (lax.broadcasted_iota(jnp.int32, (ROWS, 8), 0) * 8
         + lax.broadcasted_iota(jnp.int32, (ROWS, 8), 1))

    def center(gg):
        u = lax.shift_left(gg + OFF, 15) | (1 << 14)
        return jnp.where(gg < 1, 0.0, lax.bitcast_convert_type(u, jnp.float32))

    da = center(g) - center(g - 1)
    s = jnp.sum(jac * da)
    out_ref[...] = jnp.full((1, 1, 128), s, jnp.float32)


def kernel(logits, labels):
    lg = logits.reshape(NWORK * PER_W).astype(jnp.float32)
    lb = labels.reshape(NWORK * PER_W)

    mesh = plsc.VectorSubcoreMesh(core_axis_name="c", subcore_axis_name="s")
    hist = pl.kernel(
        _sc_hist_body,
        mesh=mesh,
        compiler_params=pltpu.CompilerParams(needs_layout_passes=False),
        out_type=jax.ShapeDtypeStruct((NWORK * PER_W,), jnp.int32),
        scratch_types=[
            pltpu.VMEM((2 * CHUNK,), jnp.float32),
            pltpu.VMEM((2 * CHUNK,), jnp.int32),
            pltpu.VMEM((NBINS * NLANE,), jnp.int32),
            pltpu.SemaphoreType.DMA((2,)),
        ],
    )(lg, lb)

    hist3 = hist.reshape(NWORK, ROWS, 128)
    per_img = pl.pallas_call(
        _tc_finish_body,
        grid=(NIMG,),
        in_specs=[pl.BlockSpec((4, ROWS, 128), lambda i: (i, 0, 0))],
        out_specs=pl.BlockSpec((1, 1, 128), lambda i: (i, 0, 0)),
        out_shape=jax.ShapeDtypeStruct((NIMG, 1, 128), jnp.float32),
    )(hist3)

    return jnp.sum(per_img[:, 0, 0]) / NIMG


# speedup vs baseline: 26.1835x; 1.5251x over previous
import functools

import jax
import jax.numpy as jnp
from jax import lax
from jax.experimental import pallas as pl
from jax.experimental.pallas import tpu as pltpu
from jax.experimental.pallas import tpu_sc as plsc

NIMG = 8
PIX = 512 * 512
NWORK = 32
PER_W = NIMG * PIX // NWORK
CHUNK = 8192
NBINS = 4096
NLANE = 16
OFF = (115 << 8) - 1
ROWS = NBINS * NLANE // 128


UNROLL = 4


def _sc_hist_body(lg_hbm, lb_hbm, out_hbm, lbuf, abuf, hist, sems):
    wid = lax.axis_index("c") * 16 + lax.axis_index("s")
    lanes = lax.iota(jnp.int32, NLANE)
    base = wid * PER_W
    nch = PER_W // CHUNK

    def issue(c, b):
        off = base + c * CHUNK
        return (
            pltpu.async_copy(lg_hbm.at[pl.ds(off, CHUNK)],
                             lbuf.at[pl.ds(b * CHUNK, CHUNK)], sems.at[b]),
            pltpu.async_copy(lb_hbm.at[pl.ds(off, CHUNK)],
                             abuf.at[pl.ds(b * CHUNK, CHUNK)], sems.at[b]),
        )

    pending = issue(0, 0)

    @plsc.parallel_loop(0, NBINS * NLANE, step=NLANE, unroll=8)
    def _zero(j):
        hist[pl.ds(j, NLANE)] = jnp.zeros((NLANE,), jnp.int32)

    for c in range(nch):
        b = c & 1
        for h in pending:
            h.wait()
        if c + 1 < nch:
            pending = issue(c + 1, 1 - b)

        @plsc.parallel_loop(b * CHUNK, b * CHUNK + CHUNK, step=NLANE,
                            unroll=UNROLL)
        def _body(i):
            sl = pl.ds(i, NLANE)
            lg = lbuf[sl]
            lb = abuf[sl]
            e = 1.0 - lg * (2.0 * lb.astype(jnp.float32) - 1.0)
            bits = lax.bitcast_convert_type(e, jnp.int32)
            bn = jnp.minimum(lax.shift_right_logical(bits, 15) - OFF,
                             NBINS - 1)
            bn = jnp.where(e < jnp.float32(2.0 ** -12),
                           jnp.zeros((NLANE,), jnp.int32), bn)
            addr = bn * NLANE + lanes
            incr = 1 + lb * 65536
            plsc.addupdate_scatter(hist, [addr], incr)

    pltpu.sync_copy(hist, out_hbm.at[pl.ds(base, PER_W)])


def _tc_finish_body(hist_ref, out_ref):
    x = (hist_ref[0] + hist_ref[1] + hist_ref[2] + hist_ref[3])
    cnt = jnp.bitwise_and(x, 65535).astype(jnp.float32)
    pos = lax.shift_right_logical(x, 16).astype(jnp.float32)

    k = lax.broadcasted_iota(jnp.int32, (128, 8), 0)
    b8 = lax.broadcasted_iota(jnp.int32, (128, 8), 1)
    m16 = (k // NLANE == b8).astype(jnp.float32)
    dot = functools.partial(jnp.dot, precision=jax.lax.Precision.HIGHEST,
                            preferred_element_type=jnp.float32)
    cnt_b = dot(cnt, m16)
    pos_b = dot(pos, m16)

    i8 = lax.broadcasted_iota(jnp.int32, (8, 8), 0)
    j8 = lax.broadcasted_iota(jnp.int32, (8, 8), 1)
    l8 = (i8 <= j8).astype(jnp.float32)
    rr = lax.broadcasted_iota(jnp.int32, (ROWS, ROWS), 0)
    cc = lax.broadcasted_iota(jnp.int32, (ROWS, ROWS), 1)
    estrict = (cc < rr).astype(jnp.float32)
    ones8 = jnp.ones((8, 1), jnp.float32)

    def cum(z):
        within = dot(z, l8)
        totals = dot(z, ones8)
        before = dot(estrict, totals)
        return within + before

    casc = cum(cnt_b)
    pasc = cum(pos_b)
    total = jnp.sum(cnt_b)
    gsum = jnp.sum(pos_b)

    n_b = total - casc + cnt_b
    cg_b = gsum - pasc + pos_b
    denom = jnp.maximum(gsum + n_b - cg_b, 1.0)
    jac = jnp.where(n_b > 0, 1.0 - (gsum - cg_b) / denom, 0.0)

    g = (lax.broadcasted_iota(jnp.int32, (ROWS, 8), 0) * 8
         + lax.broadcasted_iota(jnp.int32, (ROWS, 8), 1))

    def center(gg):
        u = lax.shift_left(gg + OFF, 15) | (1 << 14)
        return jnp.where(gg < 1, 0.0, lax.bitcast_convert_type(u, jnp.float32))

    da = center(g) - center(g - 1)
    s = jnp.sum(jac * da)
    out_ref[...] = jnp.full((1, 1, 128), s, jnp.float32)


def kernel(logits, labels):
    lg = logits.reshape(NWORK * PER_W).astype(jnp.float32)
    lb = labels.reshape(NWORK * PER_W)

    mesh = plsc.VectorSubcoreMesh(core_axis_name="c", subcore_axis_name="s")
    hist = pl.kernel(
        _sc_hist_body,
        mesh=mesh,
        compiler_params=pltpu.CompilerParams(needs_layout_passes=False),
        out_type=jax.ShapeDtypeStruct((NWORK * PER_W,), jnp.int32),
        scratch_types=[
            pltpu.VMEM((2 * CHUNK,), jnp.float32),
            pltpu.VMEM((2 * CHUNK,), jnp.int32),
            pltpu.VMEM((NBINS * NLANE,), jnp.int32),
            pltpu.SemaphoreType.DMA((2,)),
        ],
    )(lg, lb)

    hist3 = hist.reshape(NWORK, ROWS, 128)
    per_img = pl.pallas_call(
        _tc_finish_body,
        grid=(NIMG,),
        in_specs=[pl.BlockSpec((4, ROWS, 128), lambda i: (i, 0, 0))],
        out_specs=pl.BlockSpec((1, 1, 128), lambda i: (i, 0, 0)),
        out_shape=jax.ShapeDtypeStruct((NIMG, 1, 128), jnp.float32),
    )(hist3)

    return jnp.sum(per_img[:, 0, 0]) / NIMG


# speedup vs baseline: 36.6822x; 1.4010x over previous
import jax
import jax.numpy as jnp
from jax import lax
from jax.experimental import pallas as pl
from jax.experimental.pallas import tpu as pltpu
from jax.experimental.pallas import tpu_sc as plsc

NIMG = 8
PIX = 512 * 512
NWORK = 32
PER_W = NIMG * PIX // NWORK
CHUNK = 8192
NBINS = 4096
NLANE = 16
OFF = (115 << 8) - 1
OUT_W = 2 * NBINS
UNROLL = 4


def _sc_hist_body(lg_hbm, lb_hbm, out_hbm, lbuf, abuf, hist, obuf, sems):
    wid = lax.axis_index("c") * 16 + lax.axis_index("s")
    region = lax.iota(jnp.int32, NLANE) * NBINS
    base = wid * PER_W
    nch = PER_W // CHUNK

    def issue(c, b):
        off = base + c * CHUNK
        return (
            pltpu.async_copy(lg_hbm.at[pl.ds(off, CHUNK)],
                             lbuf.at[pl.ds(b * CHUNK, CHUNK)], sems.at[b]),
            pltpu.async_copy(lb_hbm.at[pl.ds(off, CHUNK)],
                             abuf.at[pl.ds(b * CHUNK, CHUNK)], sems.at[b]),
        )

    pending = issue(0, 0)

    @plsc.parallel_loop(0, NBINS * NLANE, step=NLANE, unroll=8)
    def _zero(j):
        hist[pl.ds(j, NLANE)] = jnp.zeros((NLANE,), jnp.int32)

    for c in range(nch):
        b = c & 1
        for h in pending:
            h.wait()
        if c + 1 < nch:
            pending = issue(c + 1, 1 - b)

        @plsc.parallel_loop(b * CHUNK, b * CHUNK + CHUNK, step=NLANE,
                            unroll=UNROLL)
        def _body(i):
            sl = pl.ds(i, NLANE)
            lg = lbuf[sl]
            lb = abuf[sl]
            e = 1.0 - lg * (2.0 * lb.astype(jnp.float32) - 1.0)
            bits = lax.bitcast_convert_type(e, jnp.int32)
            bn = jnp.minimum(lax.shift_right_logical(bits, 15) - OFF,
                             NBINS - 1)
            bn = jnp.where(e < jnp.float32(2.0 ** -12),
                           jnp.zeros((NLANE,), jnp.int32), bn)
            plsc.addupdate_scatter(hist, [region + bn], 1 + lb * 65536)

    @plsc.parallel_loop(0, NBINS, step=NLANE, unroll=2)
    def _reduce(j):
        v = hist[pl.ds(j, NLANE)]
        acc_c = jnp.bitwise_and(v, 65535)
        acc_p = lax.shift_right_logical(v, 16)
        for r in range(1, NLANE):
            v = hist[pl.ds(r * NBINS + j, NLANE)]
            acc_c = acc_c + jnp.bitwise_and(v, 65535)
            acc_p = acc_p + lax.shift_right_logical(v, 16)
        obuf[pl.ds(j, NLANE)] = acc_c
        obuf[pl.ds(NBINS + j, NLANE)] = acc_p

    pltpu.sync_copy(obuf, out_hbm.at[pl.ds(wid * OUT_W, OUT_W)])


def _tc_finish_body(hist_ref, out_ref):
    dot = lambda a, b: jnp.dot(a, b, precision=jax.lax.Precision.HIGHEST,
                               preferred_element_type=jnp.float32)
    i128 = lax.broadcasted_iota(jnp.int32, (128, 128), 0)
    j128 = lax.broadcasted_iota(jnp.int32, (128, 128), 1)
    l128 = (i128 <= j128).astype(jnp.float32)
    i32_ = lax.broadcasted_iota(jnp.int32, (32, 32), 0)
    j32_ = lax.broadcasted_iota(jnp.int32, (32, 32), 1)
    e32 = (j32_ < i32_).astype(jnp.float32)
    ones128 = jnp.ones((128, 1), jnp.float32)

    g = (lax.broadcasted_iota(jnp.int32, (32, 128), 0) * 128
         + lax.broadcasted_iota(jnp.int32, (32, 128), 1))

    def center(gg):
        u = lax.shift_left(gg + OFF, 15) | (1 << 14)
        return jnp.where(gg < 1, 0.0, lax.bitcast_convert_type(u, jnp.float32))

    da = center(g) - center(g - 1)

    def cum(z):
        within = dot(z, l128)
        totals = dot(z, ones128)
        return within + dot(e32, totals), totals

    total = jnp.float32(0.0)
    for i in range(NIMG):
        xw = (hist_ref[4 * i] + hist_ref[4 * i + 1]
              + hist_ref[4 * i + 2] + hist_ref[4 * i + 3])
        cnt = xw[0:32].astype(jnp.float32)
        pos = xw[32:64].astype(jnp.float32)
        casc, tc_ = cum(cnt)
        pasc, tp_ = cum(pos)
        tsum = jnp.sum(tc_)
        gsum = jnp.sum(tp_)
        n_b = tsum - casc + cnt
        cg_b = gsum - pasc + pos
        denom = jnp.maximum(gsum + n_b - cg_b, 1.0)
        jac = jnp.where(n_b > 0, 1.0 - (gsum - cg_b) / denom, 0.0)
        total = total + jnp.sum(jac * da)

    out_ref[...] = jnp.full((8, 128), total / NIMG, jnp.float32)


def kernel(logits, labels):
    lg = logits.reshape(NWORK * PER_W).astype(jnp.float32)
    lb = labels.reshape(NWORK * PER_W)

    mesh = plsc.VectorSubcoreMesh(core_axis_name="c", subcore_axis_name="s")
    hist = pl.kernel(
        _sc_hist_body,
        mesh=mesh,
        compiler_params=pltpu.CompilerParams(needs_layout_passes=False),
        out_type=jax.ShapeDtypeStruct((NWORK * OUT_W,), jnp.int32),
        scratch_types=[
            pltpu.VMEM((2 * CHUNK,), jnp.float32),
            pltpu.VMEM((2 * CHUNK,), jnp.int32),
            pltpu.VMEM((NBINS * NLANE,), jnp.int32),
            pltpu.VMEM((OUT_W,), jnp.int32),
            pltpu.SemaphoreType.DMA((2,)),
        ],
    )(lg, lb)

    hist3 = hist.reshape(NWORK, OUT_W // 128, 128)
    out = pl.pallas_call(
        _tc_finish_body,
        out_shape=jax.ShapeDtypeStruct((8, 128), jnp.float32),
    )(hist3)

    return out[0, 0]


# speedup vs baseline: 46.0157x; 1.2544x over previous
import jax
import jax.numpy as jnp
from jax import lax
from jax.experimental import pallas as pl
from jax.experimental.pallas import tpu as pltpu
from jax.experimental.pallas import tpu_sc as plsc

NIMG = 8
PIX = 512 * 512
NWORK = 32
PER_W = NIMG * PIX // NWORK
CHUNK = 8192
NBINS = 4096
NLANE = 16
OFF = (115 << 8) - 1
OUT_W = 2 * NBINS
UNROLL = 4


ROWS_PER_CHUNK = CHUNK // 512


def _sc_hist_body(lg_hbm, lb_hbm, out_hbm, lbuf, abuf, hist, obuf, sems):
    wid = lax.axis_index("c") * 16 + lax.axis_index("s")
    region = lax.iota(jnp.int32, NLANE) * NBINS
    img = wid // 4
    row0 = (wid % 4) * 128
    nch = PER_W // CHUNK

    def issue(c, b):
        rs = row0 + c * ROWS_PER_CHUNK
        dst = pl.ds(b * ROWS_PER_CHUNK, ROWS_PER_CHUNK)
        return (
            pltpu.async_copy(lg_hbm.at[img, pl.ds(rs, ROWS_PER_CHUNK), :],
                             lbuf.at[dst, :], sems.at[b]),
            pltpu.async_copy(lb_hbm.at[img, pl.ds(rs, ROWS_PER_CHUNK), :],
                             abuf.at[dst, :], sems.at[b]),
        )

    pending = issue(0, 0)

    @plsc.parallel_loop(0, NBINS * NLANE, step=NLANE, unroll=8)
    def _zero(j):
        hist[pl.ds(j, NLANE)] = jnp.zeros((NLANE,), jnp.int32)

    for c in range(nch):
        b = c & 1
        for h in pending:
            h.wait()
        if c + 1 < nch:
            pending = issue(c + 1, 1 - b)

        @plsc.parallel_loop(0, CHUNK, step=NLANE, unroll=UNROLL)
        def _body(i):
            k = i + lax.iota(jnp.int32, NLANE)
            rows = b * ROWS_PER_CHUNK + lax.shift_right_logical(k, 9)
            cols = jnp.bitwise_and(k, 511)
            lg = plsc.load_gather(lbuf, [rows, cols])
            lb = plsc.load_gather(abuf, [rows, cols])
            e = 1.0 - lg * (2.0 * lb.astype(jnp.float32) - 1.0)
            bits = lax.bitcast_convert_type(e, jnp.int32)
            bn = jnp.minimum(lax.shift_right_logical(bits, 15) - OFF,
                             NBINS - 1)
            bn = jnp.where(e < jnp.float32(2.0 ** -12),
                           jnp.zeros((NLANE,), jnp.int32), bn)
            plsc.addupdate_scatter(hist, [region + bn], 1 + lb * 65536)

    @plsc.parallel_loop(0, NBINS, step=NLANE, unroll=2)
    def _reduce(j):
        v = hist[pl.ds(j, NLANE)]
        acc_c = jnp.bitwise_and(v, 65535)
        acc_p = lax.shift_right_logical(v, 16)
        for r in range(1, NLANE):
            v = hist[pl.ds(r * NBINS + j, NLANE)]
            acc_c = acc_c + jnp.bitwise_and(v, 65535)
            acc_p = acc_p + lax.shift_right_logical(v, 16)
        obuf[pl.ds(j, NLANE)] = acc_c
        obuf[pl.ds(NBINS + j, NLANE)] = acc_p

    pltpu.sync_copy(obuf, out_hbm.at[pl.ds(wid * OUT_W, OUT_W)])


def _tc_finish_body(hist_ref, out_ref):
    dot = lambda a, b: jnp.dot(a, b, precision=jax.lax.Precision.HIGHEST,
                               preferred_element_type=jnp.float32)
    i128 = lax.broadcasted_iota(jnp.int32, (128, 128), 0)
    j128 = lax.broadcasted_iota(jnp.int32, (128, 128), 1)
    l128 = (i128 <= j128).astype(jnp.float32)
    i32_ = lax.broadcasted_iota(jnp.int32, (32, 32), 0)
    j32_ = lax.broadcasted_iota(jnp.int32, (32, 32), 1)
    e32 = (j32_ < i32_).astype(jnp.float32)
    ones128 = jnp.ones((128, 1), jnp.float32)

    g = (lax.broadcasted_iota(jnp.int32, (32, 128), 0) * 128
         + lax.broadcasted_iota(jnp.int32, (32, 128), 1))

    def center(gg):
        u = lax.shift_left(gg + OFF, 15) | (1 << 14)
        return jnp.where(gg < 1, 0.0, lax.bitcast_convert_type(u, jnp.float32))

    da = center(g) - center(g - 1)

    def cum(z):
        within = dot(z, l128)
        totals = dot(z, ones128)
        return within + dot(e32, totals), totals

    total = jnp.float32(0.0)
    for i in range(NIMG):
        xw = (hist_ref[4 * i] + hist_ref[4 * i + 1]
              + hist_ref[4 * i + 2] + hist_ref[4 * i + 3])
        cnt = xw[0:32].astype(jnp.float32)
        pos = xw[32:64].astype(jnp.float32)
        casc, tc_ = cum(cnt)
        pasc, tp_ = cum(pos)
        tsum = jnp.sum(tc_)
        gsum = jnp.sum(tp_)
        n_b = tsum - casc + cnt
        cg_b = gsum - pasc + pos
        denom = jnp.maximum(gsum + n_b - cg_b, 1.0)
        jac = jnp.where(n_b > 0, 1.0 - (gsum - cg_b) / denom, 0.0)
        total = total + jnp.sum(jac * da)

    out_ref[...] = jnp.full((8, 128), total / NIMG, jnp.float32)


def kernel(logits, labels):
    lg = logits.astype(jnp.float32)
    lb = labels

    mesh = plsc.VectorSubcoreMesh(core_axis_name="c", subcore_axis_name="s")
    hist = pl.kernel(
        _sc_hist_body,
        mesh=mesh,
        compiler_params=pltpu.CompilerParams(needs_layout_passes=False),
        out_type=jax.ShapeDtypeStruct((NWORK * OUT_W,), jnp.int32),
        scratch_types=[
            pltpu.VMEM((2 * ROWS_PER_CHUNK, 512), jnp.float32),
            pltpu.VMEM((2 * ROWS_PER_CHUNK, 512), jnp.int32),
            pltpu.VMEM((NBINS * NLANE,), jnp.int32),
            pltpu.VMEM((OUT_W,), jnp.int32),
            pltpu.SemaphoreType.DMA((2,)),
        ],
    )(lg, lb)

    hist3 = hist.reshape(NWORK, OUT_W // 128, 128)
    out = pl.pallas_call(
        _tc_finish_body,
        out_shape=jax.ShapeDtypeStruct((8, 128), jnp.float32),
    )(hist3)

    return out[0, 0]
